# baseline (speedup 1.0000x reference)
import jax
import jax.numpy as jnp
import numpy as np
from jax import lax
from jax.experimental import pallas as pl
from jax.experimental.pallas import tpu as pltpu
from jax.experimental.pallas import tpu_sc as plsc

N_IN = 512
N_BIAS = 1
N_OUT = 256
K = 16
HIDDEN = 7033
N_LEVELS = 8
N_COMPUTE = 7 * HIDDEN + N_OUT
N_NODES = N_IN + N_BIAS + N_COMPUTE
BATCH = 64

NS = 16
LANES = 16
HV = BATCH // LANES

G = 16
NT_H = 448
NT_O = N_OUT // NS
WIN = NT_H * K
WROWS = WIN // 128
N_ALLOC = N_NODES


def _body(xh, pids_l, w_l, a2, pid_t, w_t, rows_a, rows_b, out_v,
          ones_v, sem_a, sem_b):
    t = lax.axis_index("s")

    pltpu.sync_copy(xh.at[pl.ds(t * 32, 32)], a2.at[pl.ds(t * 32, 32)])
    for h in range(HV):
        ones_v[h] = jnp.ones((LANES,), jnp.float32)

    @pl.when(t == 0)
    def _():
        pltpu.sync_copy(ones_v, a2.at[N_IN])

    plsc.subcore_barrier()

    def gather(i, buf, sem):
        idx = pid_t.at[pl.ds(i * G * K, G * K)]
        return pltpu.make_async_copy(a2.at[idx], buf, sem)

    def level(lvl, _):
        is_out = lvl == N_LEVELS - 1
        nt = jnp.where(is_out, NT_O, NT_H)
        nch = nt // G
        win = (lvl * NS + t)
        dst0 = N_IN + N_BIAS + lvl * HIDDEN + t * nt

        pltpu.sync_copy(pids_l.at[pl.ds(win * WIN, WIN)], pid_t)
        pltpu.sync_copy(w_l.at[pl.ds(win * WIN, WIN)], w_t)

        def compute(i, buf):
            for g in range(G):
                w_vec = w_t[pl.ds((i * G + g) * K, K)]
                for h in range(HV):
                    acc = buf[g * K, h] * w_vec[0]
                    for k in range(1, K):
                        acc = acc + buf[g * K + k, h] * w_vec[k]
                    act = 1.0 - 2.0 / (jnp.exp(acc * 2.0) + 1.0)
                    out_v[g, h] = jnp.where(is_out, acc, act)
            pltpu.sync_copy(out_v, a2.at[pl.ds(dst0 + i * G, G)])

        gather(0, rows_a, sem_a).start()

        def pair(j, _):
            i0 = 2 * j
            gather(i0 + 1, rows_b, sem_b).start()
            gather(i0, rows_a, sem_a).wait()
            compute(i0, rows_a)

            @pl.when(i0 + 2 < nch)
            def _():
                gather(i0 + 2, rows_a, sem_a).start()

            gather(i0 + 1, rows_b, sem_b).wait()
            compute(i0 + 1, rows_b)
            return 0

        lax.fori_loop(0, nch // 2, pair, 0)

        @pl.when(nch % 2 == 1)
        def _():
            gather(nch - 1, rows_a, sem_a).wait()
            compute(nch - 1, rows_a)

        plsc.subcore_barrier()
        return 0

    lax.fori_loop(0, N_LEVELS, level, 0)


@jax.jit
def _run(xh, pids_l, w_l):
    kern = pl.kernel(
        _body,
        out_type=jax.ShapeDtypeStruct((N_ALLOC, HV, LANES), jnp.float32),
        mesh=plsc.VectorSubcoreMesh(
            core_axis_name="c", subcore_axis_name="s",
            num_cores=1, num_subcores=NS),
        compiler_params=pltpu.CompilerParams(use_tc_tiling_on_sc=False),
        scratch_types=[
            pltpu.VMEM((WIN,), jnp.int32),
            pltpu.VMEM((WIN,), jnp.float32),
            pltpu.VMEM((G * K, HV, LANES), jnp.float32),
            pltpu.VMEM((G * K, HV, LANES), jnp.float32),
            pltpu.VMEM((G, HV, LANES), jnp.float32),
            pltpu.VMEM((HV, LANES), jnp.float32),
            pltpu.SemaphoreType.DMA,
            pltpu.SemaphoreType.DMA,
        ],
    )
    return kern(xh, pids_l, w_l)


_STARTS = np.empty((N_LEVELS, NS), np.int32)
for _l in range(N_LEVELS):
    _nt = NT_O if _l == N_LEVELS - 1 else NT_H
    for _t in range(NS):
        _STARTS[_l, _t] = (_l * HIDDEN + _t * _nt) * K
_PAD_LEN = int(_STARTS.max()) + WIN - N_COMPUTE * K
_WIDX = _STARTS[:, :, None] + np.arange(WIN, dtype=np.int32)[None, None, :]


def kernel(x, weights, parent_ids):
    if x.ndim == 1:
        x = x[None, :]
    xh = x.T.reshape(N_IN, HV, LANES)
    pidsf = jnp.pad(parent_ids.astype(jnp.int32).reshape(-1), (0, _PAD_LEN))
    wf = jnp.pad(weights.reshape(-1), (0, _PAD_LEN))
    pids_l = pidsf[_WIDX].reshape(-1)
    w_l = wf[_WIDX].reshape(-1)
    a2 = _run(xh, pids_l, w_l)
    tail = a2[N_NODES - N_OUT:N_NODES]
    return tail.reshape(N_OUT, BATCH).T


# speedup vs baseline: 1.2323x; 1.2323x over previous
import jax
import jax.numpy as jnp
from jax import lax
from jax.experimental import pallas as pl
from jax.experimental.pallas import tpu as pltpu
from jax.experimental.pallas import tpu_sc as plsc

N_IN = 512
N_BIAS = 1
N_OUT = 256
K = 16
HIDDEN = 7033
N_LEVELS = 8
N_COMPUTE = 7 * HIDDEN + N_OUT
N_NODES = N_IN + N_BIAS + N_COMPUTE
BATCH = 64

NS = 16
LANES = 16
HB = BATCH // (2 * LANES)
_ILV = plsc.PackFormat.INTERLEAVED

G = 8
NT_H = 440
NT_O = N_OUT // NS


def _body(xh, pidsf, w, a2, pid_t, w_t, rows_a, rows_b, out_v, ones_v,
          sem_a, sem_b):
    t = lax.axis_index("s")

    pltpu.sync_copy(xh.at[pl.ds(t * 32, 32)], a2.at[pl.ds(t * 32, 32)])
    for h in range(HB):
        ones_v[h] = jnp.ones((2 * LANES,), jnp.bfloat16)

    @pl.when(t == 0)
    def _():
        pltpu.sync_copy(ones_v, a2.at[N_IN])

    plsc.subcore_barrier()

    def gather(base, i, buf, sem):
        idx = pid_t.at[pl.ds(base + i * G * K, G * K)]
        return pltpu.make_async_copy(a2.at[idx], buf, sem)

    def level(lvl, _):
        is_out = lvl == N_LEVELS - 1
        nt = jnp.where(is_out, NT_O, NT_H)
        nch = nt // G
        prow0 = lvl * HIDDEN + t * nt
        dst0 = N_IN + N_BIAS + lvl * HIDDEN + t * nt

        off = jnp.minimum(prow0 * K, N_COMPUTE * K - NT_H * K)
        base = prow0 * K - off
        pltpu.sync_copy(pidsf.at[pl.ds(off, NT_H * K)],
                        pid_t.at[pl.ds(0, NT_H * K)])
        pltpu.sync_copy(w.at[pl.ds(off, NT_H * K)],
                        w_t.at[pl.ds(0, NT_H * K)])

        def compute(i, buf):
            for g in range(G):
                w_vec = w_t[pl.ds(base + (i * G + g) * K, K)]
                for h in range(HB):
                    va, vb = plsc.unpack(buf[g * K, h], format=_ILV)
                    acc_a = va * w_vec[0]
                    acc_b = vb * w_vec[0]
                    for k in range(1, K):
                        va, vb = plsc.unpack(buf[g * K + k, h], format=_ILV)
                        acc_a = acc_a + va * w_vec[k]
                        acc_b = acc_b + vb * w_vec[k]
                    act_a = 1.0 - 2.0 / (jnp.exp(acc_a * 2.0) + 1.0)
                    act_b = 1.0 - 2.0 / (jnp.exp(acc_b * 2.0) + 1.0)
                    ra = jnp.where(is_out, acc_a, act_a)
                    rb = jnp.where(is_out, acc_b, act_b)
                    out_v[g, h] = plsc.pack(ra, rb, format=_ILV)
            pltpu.sync_copy(out_v, a2.at[pl.ds(dst0 + i * G, G)])

        gather(base, 0, rows_a, sem_a).start()

        def pair(j, _):
            i0 = 2 * j
            gather(base, i0 + 1, rows_b, sem_b).start()
            gather(base, i0, rows_a, sem_a).wait()
            compute(i0, rows_a)

            @pl.when(i0 + 2 < nch)
            def _():
                gather(base, i0 + 2, rows_a, sem_a).start()

            gather(base, i0 + 1, rows_b, sem_b).wait()
            compute(i0 + 1, rows_b)
            return 0

        lax.fori_loop(0, nch // 2, pair, 0)

        @pl.when(nch % 2 == 1)
        def _():
            gather(base, nch - 1, rows_a, sem_a).wait()
            compute(nch - 1, rows_a)

        plsc.subcore_barrier()
        return 0

    lax.fori_loop(0, N_LEVELS, level, 0)


@jax.jit
def _run(xh, pidsf, w):
    kern = pl.kernel(
        _body,
        out_type=jax.ShapeDtypeStruct((N_NODES, HB, 2 * LANES), jnp.bfloat16),
        mesh=plsc.VectorSubcoreMesh(
            core_axis_name="c", subcore_axis_name="s",
            num_cores=1, num_subcores=NS),
        compiler_params=pltpu.CompilerParams(use_tc_tiling_on_sc=False, needs_layout_passes=False),
        scratch_types=[
            pltpu.VMEM((NT_H * K,), jnp.int32),
            pltpu.VMEM((NT_H * K,), jnp.float32),
            pltpu.VMEM((G * K, HB, 2 * LANES), jnp.bfloat16),
            pltpu.VMEM((G * K, HB, 2 * LANES), jnp.bfloat16),
            pltpu.VMEM((G, HB, 2 * LANES), jnp.bfloat16),
            pltpu.VMEM((HB, 2 * LANES), jnp.bfloat16),
            pltpu.SemaphoreType.DMA,
            pltpu.SemaphoreType.DMA,
        ],
    )
    return kern(xh, pidsf, w)


def kernel(x, weights, parent_ids):
    if x.ndim == 1:
        x = x[None, :]
    xh = x.T.astype(jnp.bfloat16).reshape(N_IN, HB, 2 * LANES)
    pidsf = parent_ids.astype(jnp.int32).reshape(-1)
    a2 = _run(xh, pidsf, weights.reshape(-1))
    tail = a2[N_NODES - N_OUT:]
    return tail.reshape(N_OUT, BATCH).T.astype(jnp.float32)


# speedup vs baseline: 1.5977x; 1.2965x over previous
import jax
import jax.numpy as jnp
from jax import lax
from jax.experimental import pallas as pl
from jax.experimental.pallas import tpu as pltpu
from jax.experimental.pallas import tpu_sc as plsc

N_IN = 512
N_BIAS = 1
N_OUT = 256
K = 16
HIDDEN = 7033
N_LEVELS = 8
N_COMPUTE = 7 * HIDDEN + N_OUT
N_NODES = N_IN + N_BIAS + N_COMPUTE
BATCH = 64

NS = 16
LANES = 16
HV = BATCH // LANES

G = 8
NT_H = 440
NT_O = N_OUT // NS


def _body(xh, pidsf, w, a2, pid_t, w_t, rows_a, rows_b, out_v, ones_v,
          sem_a, sem_b):
    t = lax.axis_index("s")

    pltpu.sync_copy(xh.at[pl.ds(t * 32, 32)], a2.at[pl.ds(t * 32, 32)])
    for h in range(HV):
        ones_v[h] = jnp.ones((LANES,), jnp.float32)

    @pl.when(t == 0)
    def _():
        pltpu.sync_copy(ones_v, a2.at[N_IN])

    plsc.subcore_barrier()

    def gather(base, i, buf, sem):
        idx = pid_t.at[pl.ds(base + i * G * K, G * K)]
        return pltpu.make_async_copy(a2.at[idx], buf, sem)

    def level(lvl, _):
        is_out = lvl == N_LEVELS - 1
        nt = jnp.where(is_out, NT_O, NT_H)
        nch = nt // G
        prow0 = lvl * HIDDEN + t * nt
        dst0 = N_IN + N_BIAS + lvl * HIDDEN + t * nt

        off = jnp.minimum(prow0 * K, N_COMPUTE * K - NT_H * K)
        base = prow0 * K - off
        pltpu.sync_copy(pidsf.at[pl.ds(off, NT_H * K)],
                        pid_t.at[pl.ds(0, NT_H * K)])
        pltpu.sync_copy(w.at[pl.ds(off, NT_H * K)],
                        w_t.at[pl.ds(0, NT_H * K)])

        def compute(i, buf):
            for g in range(G):
                w_vec = w_t[pl.ds(base + (i * G + g) * K, K)]
                for h in range(HV):
                    acc = buf[g * K, h] * w_vec[0]
                    for k in range(1, K):
                        acc = acc + buf[g * K + k, h] * w_vec[k]
                    act = 1.0 - 2.0 / (jnp.exp(acc * 2.0) + 1.0)
                    out_v[g, h] = jnp.where(is_out, acc, act)
            pltpu.sync_copy(out_v, a2.at[pl.ds(dst0 + i * G, G)])

        gather(base, 0, rows_a, sem_a).start()

        def pair(j, _):
            i0 = 2 * j
            gather(base, i0 + 1, rows_b, sem_b).start()
            gather(base, i0, rows_a, sem_a).wait()
            compute(i0, rows_a)

            @pl.when(i0 + 2 < nch)
            def _():
                gather(base, i0 + 2, rows_a, sem_a).start()

            gather(base, i0 + 1, rows_b, sem_b).wait()
            compute(i0 + 1, rows_b)
            return 0

        lax.fori_loop(0, nch // 2, pair, 0)

        @pl.when(nch % 2 == 1)
        def _():
            gather(base, nch - 1, rows_a, sem_a).wait()
            compute(nch - 1, rows_a)

        plsc.subcore_barrier()
        return 0

    lax.fori_loop(0, N_LEVELS, level, 0)


@jax.jit
def _run(xh, pidsf, w):
    kern = pl.kernel(
        _body,
        out_type=jax.ShapeDtypeStruct((N_NODES, HV, LANES), jnp.float32),
        mesh=plsc.VectorSubcoreMesh(
            core_axis_name="c", subcore_axis_name="s",
            num_cores=1, num_subcores=NS),
        compiler_params=pltpu.CompilerParams(use_tc_tiling_on_sc=False),
        scratch_types=[
            pltpu.VMEM((NT_H * K,), jnp.int32),
            pltpu.VMEM((NT_H * K,), jnp.float32),
            pltpu.VMEM((G * K, HV, LANES), jnp.float32),
            pltpu.VMEM((G * K, HV, LANES), jnp.float32),
            pltpu.VMEM((G, HV, LANES), jnp.float32),
            pltpu.VMEM((HV, LANES), jnp.float32),
            pltpu.SemaphoreType.DMA,
            pltpu.SemaphoreType.DMA,
        ],
    )
    return kern(xh, pidsf, w)


def kernel(x, weights, parent_ids):
    if x.ndim == 1:
        x = x[None, :]
    xh = x.T.reshape(N_IN, HV, LANES)
    pidsf = parent_ids.astype(jnp.int32).reshape(-1)
    a2 = _run(xh, pidsf, weights.reshape(-1))
    tail = a2[N_NODES - N_OUT:]
    return tail.reshape(N_OUT, BATCH).T
